# baseline (speedup 1.0000x reference)
import functools

import jax
import jax.numpy as jnp
from jax import lax
from jax.experimental import pallas as pl
from jax.experimental.pallas import tpu as pltpu
from jax.experimental.pallas import tpu_sc as plsc

N_NODES = 10000
N_REL = 16
N_EDGES = 320000
N_BASIS = 8

NC = 2
NS = 16
NW = NC * NS

E_PER_W = N_EDGES // NW
BLK_E = 2000
N_BLK_E = E_PER_W // BLK_E
GRP = 80
N_GRP = BLK_E // GRP
ROWS_PER_TEC = N_NODES // NS
ZROWS = 125

LANE = 16


def _mesh():
    return plsc.VectorSubcoreMesh(core_axis_name="c", subcore_axis_name="s")


N_GRP_W = E_PER_W // GRP


def _build_keys(hi_v, lo_v, key2d_v, mult, n_groups):
    def body(i, _):
        for j in range(GRP // LANE):
            off = i * GRP + j * LANE
            key2d_v[i, pl.ds(j * LANE, LANE)] = (
                hi_v[pl.ds(off, LANE)] * mult + lo_v[pl.ds(off, LANE)])
        return 0

    lax.fori_loop(0, n_groups, body, 0)


E_PER_TILE = N_EDGES // NS
N_GRP_T = E_PER_TILE // GRP


def _counts_inv(dst, etype):
    kern = pl.kernel(
        _counts_inv_body,
        out_type=jax.ShapeDtypeStruct((N_EDGES,), jnp.float32),
        mesh=_mesh(),
        scratch_types=dict(
            counts_sp=pltpu.VMEM_SHARED((N_REL * N_NODES,), jnp.float32),
            et_v=pltpu.VMEM((E_PER_TILE,), jnp.int32),
            dst_v=pltpu.VMEM((E_PER_TILE,), jnp.int32),
            key2d_v=pltpu.VMEM((N_GRP_T, GRP), jnp.int32),
            ones_v=pltpu.VMEM((GRP,), jnp.float32),
            cv0=pltpu.VMEM((GRP,), jnp.float32),
            cv1=pltpu.VMEM((GRP,), jnp.float32),
            ibuf_v=pltpu.VMEM((E_PER_W,), jnp.float32),
            zbuf_v=pltpu.VMEM((BLK_E,), jnp.float32),
            sem_s=pltpu.SemaphoreType.DMA,
            sem_g0=pltpu.SemaphoreType.DMA,
            sem_g1=pltpu.SemaphoreType.DMA,
        ),
        compiler_params=pltpu.CompilerParams(use_tc_tiling_on_sc=False),
    )
    return kern(dst, etype)


def _counts_inv_body(dst_hbm, et_hbm, inv_hbm,
                     counts_sp, et_v, dst_v, key2d_v, ones_v, cv0, cv1,
                     ibuf_v, zbuf_v, sem_s, sem_g0, sem_g1):
    cid = lax.axis_index("c")
    sid = lax.axis_index("s")
    wid = sid * NC + cid

    z16 = jnp.zeros((LANE,), jnp.float32)
    o16 = jnp.ones((LANE,), jnp.float32)
    for g in range(GRP // LANE):
        ones_v[pl.ds(g * LANE, LANE)] = o16

    def zero_blk(i, _):
        zbuf_v[pl.ds(i * LANE, LANE)] = z16
        return 0

    lax.fori_loop(0, BLK_E // LANE, zero_blk, 0)
    for t in range(N_REL * N_NODES // NS // BLK_E):
        pltpu.sync_copy(zbuf_v,
                        counts_sp.at[pl.ds(sid * (N_REL * N_NODES // NS)
                                           + t * BLK_E, BLK_E)])
    plsc.subcore_barrier()

    base1 = sid * E_PER_TILE
    pltpu.sync_copy(et_hbm.at[pl.ds(base1, E_PER_TILE)], et_v)
    pltpu.sync_copy(dst_hbm.at[pl.ds(base1, E_PER_TILE)], dst_v)
    _build_keys(et_v, dst_v, key2d_v, N_NODES, N_GRP_T)

    def fire(i, _):
        pltpu.async_copy(ones_v, counts_sp.at[key2d_v.at[i]], sem_s,
                         add=True)
        return 0

    lax.fori_loop(0, N_GRP_T, fire, 0)

    def drain(i, _):
        pltpu.make_async_copy(ones_v, counts_sp.at[key2d_v.at[i]],
                              sem_s).wait()
        return 0

    lax.fori_loop(0, N_GRP_T, drain, 0)
    plsc.subcore_barrier()

    base2 = wid * E_PER_W
    pltpu.sync_copy(et_hbm.at[pl.ds(base2, E_PER_W)],
                    et_v.at[pl.ds(0, E_PER_W)])
    pltpu.sync_copy(dst_hbm.at[pl.ds(base2, E_PER_W)],
                    dst_v.at[pl.ds(0, E_PER_W)])
    _build_keys(et_v, dst_v, key2d_v, N_NODES, N_GRP_W)
    cv = [cv0, cv1]
    sems = [sem_g0, sem_g1]

    def gather(i, p):
        return pltpu.async_copy(counts_sp.at[key2d_v.at[i]], cv[p], sems[p])

    def gwait(i, p):
        pltpu.make_async_copy(counts_sp.at[key2d_v.at[i]], cv[p],
                              sems[p]).wait()

    def process(i, p):
        for j in range(GRP // LANE):
            c = cv[p][pl.ds(j * LANE, LANE)]
            ibuf_v[pl.ds(i * GRP + j * LANE, LANE)] = (
                1.0 / jnp.maximum(c, 1.0))

    gather(0, 0)
    gather(1, 1)

    def pair(jj, _):
        for k in range(2):
            i = 2 * jj + k
            gwait(i, k)
            process(i, k)

            @pl.when(i + 2 < N_GRP_W)
            def _():
                gather(i + 2, k)
        return 0

    lax.fori_loop(0, N_GRP_W // 2, pair, 0)
    gwait(N_GRP_W - 1, 0)
    process(N_GRP_W - 1, 0)
    pltpu.sync_copy(ibuf_v, inv_hbm.at[pl.ds(base2, E_PER_W)])


def _edge_pass(table, src, etype, inv, dst, d):
    body = _edge_pass_body if d <= 16 else _edge_pass_body_blk
    kern = pl.kernel(
        functools.partial(body, d=d),
        out_type=jax.ShapeDtypeStruct((NC, NS, ROWS_PER_TEC, d), jnp.float32),
        mesh=_mesh(),
        scratch_types=dict(
            agg_sp=pltpu.VMEM_SHARED((N_NODES, d), jnp.float32),
            src_v=pltpu.VMEM((E_PER_W,), jnp.int32),
            et_v=pltpu.VMEM((E_PER_W,), jnp.int32),
            dst_v=pltpu.VMEM((E_PER_W,), jnp.int32),
            inv_v=pltpu.VMEM((E_PER_W,), jnp.float32),
            dst2d_v=pltpu.VMEM((N_GRP_W, GRP), jnp.int32),
            key2d_v=pltpu.VMEM((N_GRP_W, GRP), jnp.int32),
            rows0=pltpu.VMEM((GRP, d), jnp.float32),
            rows1=pltpu.VMEM((GRP, d), jnp.float32),
            rows2=pltpu.VMEM((GRP, d), jnp.float32),
            rows3=pltpu.VMEM((GRP, d), jnp.float32),
            zbuf_v=pltpu.VMEM((ZROWS, d), jnp.float32),
            sem_g0=pltpu.SemaphoreType.DMA,
            sem_g1=pltpu.SemaphoreType.DMA,
            sem_g2=pltpu.SemaphoreType.DMA,
            sem_g3=pltpu.SemaphoreType.DMA,
            sem_s0=pltpu.SemaphoreType.DMA,
            sem_s1=pltpu.SemaphoreType.DMA,
            sem_s2=pltpu.SemaphoreType.DMA,
            sem_s3=pltpu.SemaphoreType.DMA,
        ),
        compiler_params=pltpu.CompilerParams(use_tc_tiling_on_sc=False),
    )
    return kern(table, src, etype, inv, dst).reshape(NC, N_NODES, d)


def _edge_pass_body(table_hbm, src_hbm, et_hbm, inv_hbm, dst_hbm, out_hbm,
                    agg_sp, src_v, et_v, dst_v, inv_v, dst2d_v, key2d_v,
                    rows0, rows1, rows2, rows3, zbuf_v,
                    sem_g0, sem_g1, sem_g2, sem_g3,
                    sem_s0, sem_s1, sem_s2, sem_s3, *, d):
    cid = lax.axis_index("c")
    sid = lax.axis_index("s")
    wid = sid * NC + cid
    nk = d // LANE
    rows = [rows0, rows1, rows2, rows3]
    gsems = [sem_g0, sem_g1, sem_g2, sem_g3]
    ssems = [sem_s0, sem_s1, sem_s2, sem_s3]

    z16 = jnp.zeros((LANE,), jnp.float32)

    def zero_row(i, _):
        for k in range(nk):
            zbuf_v[i, pl.ds(k * LANE, LANE)] = z16
        return 0

    lax.fori_loop(0, ZROWS, zero_row, 0)
    for t in range(ROWS_PER_TEC // ZROWS):
        pltpu.sync_copy(zbuf_v,
                        agg_sp.at[pl.ds(sid * ROWS_PER_TEC + t * ZROWS,
                                        ZROWS)])
    plsc.subcore_barrier()

    base = wid * E_PER_W
    pltpu.sync_copy(src_hbm.at[pl.ds(base, E_PER_W)], src_v)
    pltpu.sync_copy(et_hbm.at[pl.ds(base, E_PER_W)], et_v)
    pltpu.sync_copy(inv_hbm.at[pl.ds(base, E_PER_W)], inv_v)
    pltpu.sync_copy(dst_hbm.at[pl.ds(base, E_PER_W)], dst_v)
    _build_keys(src_v, et_v, key2d_v, N_REL + 1, N_GRP_W)

    def cp2d(i, _):
        for j in range(GRP // LANE):
            off = i * GRP + j * LANE
            dst2d_v[i, pl.ds(j * LANE, LANE)] = dst_v[pl.ds(off, LANE)]
        return 0

    lax.fori_loop(0, N_GRP_W, cp2d, 0)

    def scale(i, p):
        def body(g, _):
            inv16 = inv_v[pl.ds(i * GRP + g * LANE, LANE)]
            for e in range(LANE):
                s = inv16[e]
                row = g * LANE + e
                for k in range(nk):
                    rows[p][row, pl.ds(k * LANE, LANE)] = (
                        rows[p][row, pl.ds(k * LANE, LANE)] * s)
            return 0

        lax.fori_loop(0, GRP // LANE, body, 0)

    def gather(i, p):
        pltpu.async_copy(table_hbm.at[key2d_v.at[i]], rows[p], gsems[p])

    def gwait(i, p):
        pltpu.make_async_copy(table_hbm.at[key2d_v.at[i]], rows[p],
                              gsems[p]).wait()

    def scat(i, p):
        pltpu.async_copy(rows[p], agg_sp.at[dst2d_v.at[i]], ssems[p],
                         add=True)

    def swait(i, p):
        pltpu.make_async_copy(rows[p], agg_sp.at[dst2d_v.at[i]],
                              ssems[p]).wait()

    gather(0, 0)
    gather(1, 1)
    for k in range(4):
        gwait(k, k)
        scale(k, k)
        scat(k, k)
        if k >= 2:
            swait(k - 2, k - 2)
        gather(k + 2, (k + 2) % 4)

    def quad(jj, _):
        for k in range(4):
            i = 4 * jj + k
            gwait(i, k)
            scale(i, k)
            scat(i, k)

            @pl.when(i + 2 < N_GRP_W)
            def _():
                swait(i - 2, (k + 2) % 4)
                gather(i + 2, (k + 2) % 4)
        return 0

    lax.fori_loop(1, N_GRP_W // 4, quad, 0)
    i = N_GRP_W - 1
    gwait(i, 0)
    scale(i, 0)
    scat(i, 0)
    for m in range(N_GRP_W - 4, N_GRP_W):
        swait(m, m % 4)

    plsc.subcore_barrier()
    pltpu.sync_copy(agg_sp.at[pl.ds(sid * ROWS_PER_TEC, ROWS_PER_TEC)],
                    out_hbm.at[cid, sid])


def _edge_pass_body_blk(table_hbm, src_hbm, et_hbm, inv_hbm, dst_hbm, out_hbm,
                        agg_sp, src_v, et_v, dst_v, inv_v, dst2d_v, key2d_v,
                        rows0, rows1, rows2, rows3, zbuf_v,
                        sem_g0, sem_g1, sem_g2, sem_g3,
                        sem_s0, sem_s1, sem_s2, sem_s3, *, d):
    cid = lax.axis_index("c")
    sid = lax.axis_index("s")
    wid = sid * NC + cid
    nk = d // LANE
    rows = [rows0, rows1, rows2, rows3]
    gsems = [sem_g0, sem_g1, sem_g2, sem_g3]
    ssems = [sem_s0, sem_s1, sem_s2, sem_s3]

    z16 = jnp.zeros((LANE,), jnp.float32)

    def zero_row(i, _):
        for k in range(nk):
            zbuf_v[i, pl.ds(k * LANE, LANE)] = z16
        return 0

    lax.fori_loop(0, ZROWS, zero_row, 0)
    for t in range(ROWS_PER_TEC // ZROWS):
        pltpu.sync_copy(zbuf_v,
                        agg_sp.at[pl.ds(sid * ROWS_PER_TEC + t * ZROWS,
                                        ZROWS)])
    plsc.subcore_barrier()

    base = wid * E_PER_W
    pltpu.sync_copy(src_hbm.at[pl.ds(base, E_PER_W)], src_v)
    pltpu.sync_copy(et_hbm.at[pl.ds(base, E_PER_W)], et_v)
    pltpu.sync_copy(inv_hbm.at[pl.ds(base, E_PER_W)], inv_v)
    pltpu.sync_copy(dst_hbm.at[pl.ds(base, E_PER_W)], dst_v)
    _build_keys(src_v, et_v, key2d_v, N_REL + 1, N_GRP_W)

    def cp2d(i, _):
        for j in range(GRP // LANE):
            off = i * GRP + j * LANE
            dst2d_v[i, pl.ds(j * LANE, LANE)] = dst_v[pl.ds(off, LANE)]
        return 0

    lax.fori_loop(0, N_GRP_W, cp2d, 0)

    def scale(buf, i):
        def body(g, _):
            inv16 = inv_v[pl.ds(i * GRP + g * LANE, LANE)]
            for e in range(LANE):
                s = inv16[e]
                row = g * LANE + e
                for k in range(nk):
                    buf[row, pl.ds(k * LANE, LANE)] = (
                        buf[row, pl.ds(k * LANE, LANE)] * s)
            return 0

        lax.fori_loop(0, GRP // LANE, body, 0)

    def blk(b, _):
        gbase = b * N_GRP

        def gather(i, p):
            return pltpu.async_copy(table_hbm.at[key2d_v.at[gbase + i]],
                                    rows[p], gsems[p])

        dg = [gather(0, 0), gather(1, 1), None, None]
        ds = [None, None, None, None]
        for i in range(N_GRP):
            p = i % 4
            dg[p].wait()
            scale(rows[p], gbase + i)
            ds[p] = pltpu.async_copy(rows[p],
                                     agg_sp.at[dst2d_v.at[gbase + i]],
                                     ssems[p], add=True)
            if i + 2 < N_GRP:
                q = (i + 2) % 4
                if ds[q] is not None:
                    ds[q].wait()
                    ds[q] = None
                dg[q] = gather(i + 2, q)
        for p in range(4):
            if ds[p] is not None:
                ds[p].wait()
        return 0

    lax.fori_loop(0, N_BLK_E, blk, 0)
    plsc.subcore_barrier()
    pltpu.sync_copy(agg_sp.at[pl.ds(sid * ROWS_PER_TEC, ROWS_PER_TEC)],
                    out_hbm.at[cid, sid])


BLK_N = 2000


def _dense(x, comp_aug, bases_aug):
    r, nb = comp_aug.shape
    _, cin, cout = bases_aug.shape
    n = x.shape[0]

    def body(comp_ref, bases_ref, x_ref, out_ref, wcat_ref):
        @pl.when(pl.program_id(0) == 0)
        def _build():
            for rr in range(r):
                w = comp_ref[rr, 0] * bases_ref[0]
                for b in range(1, nb):
                    w = w + comp_ref[rr, b] * bases_ref[b]
                wcat_ref[:, rr * cout:(rr + 1) * cout] = w

        out_ref[...] = jnp.dot(x_ref[...], wcat_ref[...],
                               preferred_element_type=jnp.float32)

    return pl.pallas_call(
        body,
        grid=(n // BLK_N,),
        in_specs=[
            pl.BlockSpec((r, nb), lambda j: (0, 0),
                         memory_space=pltpu.SMEM),
            pl.BlockSpec((nb, cin, cout), lambda j: (0, 0, 0)),
            pl.BlockSpec((BLK_N, cin), lambda j: (j, 0)),
        ],
        out_specs=pl.BlockSpec((BLK_N, r * cout), lambda j: (j, 0)),
        out_shape=jax.ShapeDtypeStruct((n, r * cout), jnp.float32),
        scratch_shapes=[pltpu.VMEM((cin, r * cout), jnp.float32)],
    )(comp_aug, bases_aug, x)


def _combine(parts, xt_all, bias, relu):
    _, n, cout = parts.shape

    def body(parts_ref, xr_ref, bias_ref, out_ref):
        s = (parts_ref[0] + parts_ref[1] + xr_ref[:, :cout]
             + bias_ref[...])
        out_ref[...] = jnp.maximum(s, 0.0) if relu else s

    return pl.pallas_call(
        body,
        grid=(n // BLK_N,),
        in_specs=[
            pl.BlockSpec((NC, BLK_N, cout), lambda j: (0, j, 0)),
            pl.BlockSpec((BLK_N, 128), lambda j: (j, (N_REL * cout) // 128)),
            pl.BlockSpec((1, cout), lambda j: (0, 0)),
        ],
        out_specs=pl.BlockSpec((BLK_N, cout), lambda j: (j, 0)),
        out_shape=jax.ShapeDtypeStruct((n, cout), jnp.float32),
    )(parts, xt_all, bias.reshape(1, cout))


def _augment(comp, bases, root):
    nb = comp.shape[1]
    bases_aug = jnp.concatenate([bases, root[None]], axis=0)
    comp_aug = jnp.concatenate(
        [jnp.concatenate([comp, jnp.zeros((comp.shape[0], 1), comp.dtype)],
                         axis=1),
         jnp.zeros((1, nb + 1), comp.dtype).at[0, nb].set(1.0)],
        axis=0)
    return comp_aug, bases_aug


def kernel(x, bases1, comp1, root1, bias1, bases2, comp2, root2, bias2,
           edge_index, edge_type):
    src = edge_index[0]
    dst = edge_index[1]

    inv = _counts_inv(dst, edge_type)

    ca1, ba1 = _augment(comp1, bases1, root1)
    xt1 = _dense(x, ca1, ba1)
    d1 = bases1.shape[2]
    p1 = _edge_pass(xt1.reshape(N_NODES * (N_REL + 1), d1),
                    src, edge_type, inv, dst, d1)
    h = _combine(p1, xt1, bias1, relu=True)

    ca2, ba2 = _augment(comp2, bases2, root2)
    xt2 = _dense(h, ca2, ba2)
    d2 = bases2.shape[2]
    p2 = _edge_pass(xt2.reshape(N_NODES * (N_REL + 1), d2),
                    src, edge_type, inv, dst, d2)
    return _combine(p2, xt2, bias2, relu=False)


# speedup vs baseline: 1.2254x; 1.2254x over previous
import functools

import jax
import jax.numpy as jnp
from jax import lax
from jax.experimental import pallas as pl
from jax.experimental.pallas import tpu as pltpu
from jax.experimental.pallas import tpu_sc as plsc

N_NODES = 10000
N_REL = 16
N_EDGES = 320000
N_BASIS = 8

NC = 2
NS = 16
NW = NC * NS

E_PER_W = N_EDGES // NW
BLK_E = 2000
N_BLK_E = E_PER_W // BLK_E
GRP = 80
N_GRP = BLK_E // GRP
ROWS_PER_TEC = N_NODES // NS
ZROWS = 125

LANE = 16


def _mesh():
    return plsc.VectorSubcoreMesh(core_axis_name="c", subcore_axis_name="s")


N_GRP_W = E_PER_W // GRP


def _build_keys(hi_v, lo_v, key2d_v, mult, n_groups):
    def body(i, _):
        for j in range(GRP // LANE):
            off = i * GRP + j * LANE
            key2d_v[i, pl.ds(j * LANE, LANE)] = (
                hi_v[pl.ds(off, LANE)] * mult + lo_v[pl.ds(off, LANE)])
        return 0

    lax.fori_loop(0, n_groups, body, 0)


E_PER_TILE = N_EDGES // NS
N_GRP_T = E_PER_TILE // GRP


def _counts_inv(dst, etype):
    kern = pl.kernel(
        _counts_inv_body,
        out_type=jax.ShapeDtypeStruct((N_EDGES,), jnp.float32),
        mesh=_mesh(),
        scratch_types=dict(
            counts_sp=pltpu.VMEM_SHARED((N_REL * N_NODES,), jnp.float32),
            et_v=pltpu.VMEM((E_PER_TILE,), jnp.int32),
            dst_v=pltpu.VMEM((E_PER_TILE,), jnp.int32),
            key2d_v=pltpu.VMEM((N_GRP_T, GRP), jnp.int32),
            ones_v=pltpu.VMEM((GRP,), jnp.float32),
            cv0=pltpu.VMEM((GRP,), jnp.float32),
            cv1=pltpu.VMEM((GRP,), jnp.float32),
            ibuf_v=pltpu.VMEM((E_PER_W,), jnp.float32),
            zbuf_v=pltpu.VMEM((BLK_E,), jnp.float32),
            sem_s=pltpu.SemaphoreType.DMA,
            sem_g0=pltpu.SemaphoreType.DMA,
            sem_g1=pltpu.SemaphoreType.DMA,
        ),
        compiler_params=pltpu.CompilerParams(use_tc_tiling_on_sc=False),
    )
    return kern(dst, etype)


def _counts_inv_body(dst_hbm, et_hbm, inv_hbm,
                     counts_sp, et_v, dst_v, key2d_v, ones_v, cv0, cv1,
                     ibuf_v, zbuf_v, sem_s, sem_g0, sem_g1):
    cid = lax.axis_index("c")
    sid = lax.axis_index("s")
    wid = sid * NC + cid

    z16 = jnp.zeros((LANE,), jnp.float32)
    o16 = jnp.ones((LANE,), jnp.float32)
    for g in range(GRP // LANE):
        ones_v[pl.ds(g * LANE, LANE)] = o16

    def zero_blk(i, _):
        zbuf_v[pl.ds(i * LANE, LANE)] = z16
        return 0

    lax.fori_loop(0, BLK_E // LANE, zero_blk, 0)
    for t in range(N_REL * N_NODES // NS // BLK_E):
        pltpu.sync_copy(zbuf_v,
                        counts_sp.at[pl.ds(sid * (N_REL * N_NODES // NS)
                                           + t * BLK_E, BLK_E)])
    plsc.subcore_barrier()

    base1 = sid * E_PER_TILE
    pltpu.sync_copy(et_hbm.at[pl.ds(base1, E_PER_TILE)], et_v)
    pltpu.sync_copy(dst_hbm.at[pl.ds(base1, E_PER_TILE)], dst_v)
    _build_keys(et_v, dst_v, key2d_v, N_NODES, N_GRP_T)

    def fire(i, _):
        pltpu.async_copy(ones_v, counts_sp.at[key2d_v.at[i]], sem_s,
                         add=True)
        return 0

    lax.fori_loop(0, N_GRP_T, fire, 0)

    def drain(i, _):
        pltpu.make_async_copy(ones_v, counts_sp.at[key2d_v.at[i]],
                              sem_s).wait()
        return 0

    lax.fori_loop(0, N_GRP_T, drain, 0)
    plsc.subcore_barrier()

    base2 = wid * E_PER_W
    pltpu.sync_copy(et_hbm.at[pl.ds(base2, E_PER_W)],
                    et_v.at[pl.ds(0, E_PER_W)])
    pltpu.sync_copy(dst_hbm.at[pl.ds(base2, E_PER_W)],
                    dst_v.at[pl.ds(0, E_PER_W)])
    _build_keys(et_v, dst_v, key2d_v, N_NODES, N_GRP_W)
    cv = [cv0, cv1]
    sems = [sem_g0, sem_g1]

    def gather(i, p):
        return pltpu.async_copy(counts_sp.at[key2d_v.at[i]], cv[p], sems[p])

    def gwait(i, p):
        pltpu.make_async_copy(counts_sp.at[key2d_v.at[i]], cv[p],
                              sems[p]).wait()

    def process(i, p):
        for j in range(GRP // LANE):
            c = cv[p][pl.ds(j * LANE, LANE)]
            ibuf_v[pl.ds(i * GRP + j * LANE, LANE)] = (
                1.0 / jnp.maximum(c, 1.0))

    gather(0, 0)
    gather(1, 1)

    def pair(jj, _):
        for k in range(2):
            i = 2 * jj + k
            gwait(i, k)
            process(i, k)

            @pl.when(i + 2 < N_GRP_W)
            def _():
                gather(i + 2, k)
        return 0

    lax.fori_loop(0, N_GRP_W // 2, pair, 0)
    gwait(N_GRP_W - 1, 0)
    process(N_GRP_W - 1, 0)
    pltpu.sync_copy(ibuf_v, inv_hbm.at[pl.ds(base2, E_PER_W)])


def _edge_pass(table, src, etype, inv, dst, d):
    single = d <= 16
    body = _edge_pass_body if single else _edge_pass_body_blk
    ne = E_PER_W if single else BLK_E
    ng = N_GRP_W if single else N_GRP
    kern = pl.kernel(
        functools.partial(body, d=d),
        out_type=jax.ShapeDtypeStruct((NC, NS, ROWS_PER_TEC, d), jnp.float32),
        mesh=_mesh(),
        scratch_types=dict(
            agg_sp=pltpu.VMEM_SHARED((N_NODES, d), jnp.float32),
            src_v=pltpu.VMEM((ne,), jnp.int32),
            et_v=pltpu.VMEM((ne,), jnp.int32),
            dst_v=pltpu.VMEM((ne,), jnp.int32),
            inv_v=pltpu.VMEM((ne,), jnp.float32),
            dst2d_v=pltpu.VMEM((ng, GRP), jnp.int32),
            key2d_v=pltpu.VMEM((ng, GRP), jnp.int32),
            rows0=pltpu.VMEM((GRP, d), jnp.float32),
            rows1=pltpu.VMEM((GRP, d), jnp.float32),
            rows2=pltpu.VMEM((GRP, d), jnp.float32),
            rows3=pltpu.VMEM((GRP, d), jnp.float32),
            zbuf_v=pltpu.VMEM((ZROWS, d), jnp.float32),
            sem_g0=pltpu.SemaphoreType.DMA,
            sem_g1=pltpu.SemaphoreType.DMA,
            sem_g2=pltpu.SemaphoreType.DMA,
            sem_g3=pltpu.SemaphoreType.DMA,
            sem_s0=pltpu.SemaphoreType.DMA,
            sem_s1=pltpu.SemaphoreType.DMA,
            sem_s2=pltpu.SemaphoreType.DMA,
            sem_s3=pltpu.SemaphoreType.DMA,
        ),
        compiler_params=pltpu.CompilerParams(use_tc_tiling_on_sc=False),
    )
    return kern(table, src, etype, inv, dst).reshape(NC, N_NODES, d)


def _edge_pass_body(table_hbm, src_hbm, et_hbm, inv_hbm, dst_hbm, out_hbm,
                    agg_sp, src_v, et_v, dst_v, inv_v, dst2d_v, key2d_v,
                    rows0, rows1, rows2, rows3, zbuf_v,
                    sem_g0, sem_g1, sem_g2, sem_g3,
                    sem_s0, sem_s1, sem_s2, sem_s3, *, d):
    cid = lax.axis_index("c")
    sid = lax.axis_index("s")
    wid = sid * NC + cid
    nk = d // LANE
    rows = [rows0, rows1, rows2, rows3]
    gsems = [sem_g0, sem_g1, sem_g2, sem_g3]
    ssems = [sem_s0, sem_s1, sem_s2, sem_s3]

    z16 = jnp.zeros((LANE,), jnp.float32)

    def zero_row(i, _):
        for k in range(nk):
            zbuf_v[i, pl.ds(k * LANE, LANE)] = z16
        return 0

    lax.fori_loop(0, ZROWS, zero_row, 0)
    for t in range(ROWS_PER_TEC // ZROWS):
        pltpu.sync_copy(zbuf_v,
                        agg_sp.at[pl.ds(sid * ROWS_PER_TEC + t * ZROWS,
                                        ZROWS)])
    plsc.subcore_barrier()

    base = wid * E_PER_W
    pltpu.sync_copy(src_hbm.at[pl.ds(base, E_PER_W)], src_v)
    pltpu.sync_copy(et_hbm.at[pl.ds(base, E_PER_W)], et_v)
    pltpu.sync_copy(inv_hbm.at[pl.ds(base, E_PER_W)], inv_v)
    pltpu.sync_copy(dst_hbm.at[pl.ds(base, E_PER_W)], dst_v)
    _build_keys(src_v, et_v, key2d_v, N_REL + 1, N_GRP_W)

    def cp2d(i, _):
        for j in range(GRP // LANE):
            off = i * GRP + j * LANE
            dst2d_v[i, pl.ds(j * LANE, LANE)] = dst_v[pl.ds(off, LANE)]
        return 0

    lax.fori_loop(0, N_GRP_W, cp2d, 0)

    def scale(i, p):
        def body(g, _):
            inv16 = inv_v[pl.ds(i * GRP + g * LANE, LANE)]
            for e in range(LANE):
                s = inv16[e]
                row = g * LANE + e
                for k in range(nk):
                    rows[p][row, pl.ds(k * LANE, LANE)] = (
                        rows[p][row, pl.ds(k * LANE, LANE)] * s)
            return 0

        lax.fori_loop(0, GRP // LANE, body, 0)

    def gather(i, p):
        pltpu.async_copy(table_hbm.at[key2d_v.at[i]], rows[p], gsems[p])

    def gwait(i, p):
        pltpu.make_async_copy(table_hbm.at[key2d_v.at[i]], rows[p],
                              gsems[p]).wait()

    def scat(i, p):
        pltpu.async_copy(rows[p], agg_sp.at[dst2d_v.at[i]], ssems[p],
                         add=True)

    def swait(i, p):
        pltpu.make_async_copy(rows[p], agg_sp.at[dst2d_v.at[i]],
                              ssems[p]).wait()

    gather(0, 0)
    gather(1, 1)
    for k in range(4):
        gwait(k, k)
        scale(k, k)
        scat(k, k)
        if k >= 2:
            swait(k - 2, k - 2)
        gather(k + 2, (k + 2) % 4)

    def quad(jj, _):
        for k in range(4):
            i = 4 * jj + k
            gwait(i, k)
            scale(i, k)
            scat(i, k)

            @pl.when(i + 2 < N_GRP_W)
            def _():
                swait(i - 2, (k + 2) % 4)
                gather(i + 2, (k + 2) % 4)
        return 0

    lax.fori_loop(1, N_GRP_W // 4, quad, 0)
    i = N_GRP_W - 1
    gwait(i, 0)
    scale(i, 0)
    scat(i, 0)
    for m in range(N_GRP_W - 4, N_GRP_W):
        swait(m, m % 4)

    plsc.subcore_barrier()
    pltpu.sync_copy(agg_sp.at[pl.ds(sid * ROWS_PER_TEC, ROWS_PER_TEC)],
                    out_hbm.at[cid, sid])


def _edge_pass_body_blk(table_hbm, src_hbm, et_hbm, inv_hbm, dst_hbm, out_hbm,
                        agg_sp, src_v, et_v, dst_v, inv_v, dst2d_v, key2d_v,
                        rows0, rows1, rows2, rows3, zbuf_v,
                        sem_g0, sem_g1, sem_g2, sem_g3,
                        sem_s0, sem_s1, sem_s2, sem_s3, *, d):
    cid = lax.axis_index("c")
    sid = lax.axis_index("s")
    wid = sid * NC + cid
    nk = d // LANE
    rows = [rows0, rows1, rows2, rows3]
    gsems = [sem_g0, sem_g1, sem_g2, sem_g3]
    ssems = [sem_s0, sem_s1, sem_s2, sem_s3]

    z16 = jnp.zeros((LANE,), jnp.float32)

    def zero_row(i, _):
        for k in range(nk):
            zbuf_v[i, pl.ds(k * LANE, LANE)] = z16
        return 0

    lax.fori_loop(0, ZROWS, zero_row, 0)
    for t in range(ROWS_PER_TEC // ZROWS):
        pltpu.sync_copy(zbuf_v,
                        agg_sp.at[pl.ds(sid * ROWS_PER_TEC + t * ZROWS,
                                        ZROWS)])
    plsc.subcore_barrier()

    def scale(buf, i):
        def body(g, _):
            inv16 = inv_v[pl.ds(i * GRP + g * LANE, LANE)]
            for e in range(LANE):
                s = inv16[e]
                row = g * LANE + e
                for k in range(nk):
                    buf[row, pl.ds(k * LANE, LANE)] = (
                        buf[row, pl.ds(k * LANE, LANE)] * s)
            return 0

        lax.fori_loop(0, GRP // LANE, body, 0)

    def blk(b, _):
        base = wid * E_PER_W + b * BLK_E
        pltpu.sync_copy(src_hbm.at[pl.ds(base, BLK_E)], src_v)
        pltpu.sync_copy(et_hbm.at[pl.ds(base, BLK_E)], et_v)
        pltpu.sync_copy(inv_hbm.at[pl.ds(base, BLK_E)], inv_v)
        pltpu.sync_copy(dst_hbm.at[pl.ds(base, BLK_E)], dst_v)
        for i in range(N_GRP):
            for j in range(GRP // LANE):
                off = i * GRP + j * LANE
                key2d_v[i, pl.ds(j * LANE, LANE)] = (
                    src_v[pl.ds(off, LANE)] * (N_REL + 1)
                    + et_v[pl.ds(off, LANE)])
                dst2d_v[i, pl.ds(j * LANE, LANE)] = dst_v[pl.ds(off, LANE)]

        def gather(i, p):
            return pltpu.async_copy(table_hbm.at[key2d_v.at[i]], rows[p],
                                    gsems[p])

        dg = [gather(0, 0), gather(1, 1), None, None]
        ds = [None, None, None, None]
        for i in range(N_GRP):
            p = i % 4
            dg[p].wait()
            scale(rows[p], i)
            ds[p] = pltpu.async_copy(rows[p], agg_sp.at[dst2d_v.at[i]],
                                     ssems[p], add=True)
            if i + 2 < N_GRP:
                q = (i + 2) % 4
                if ds[q] is not None:
                    ds[q].wait()
                    ds[q] = None
                dg[q] = gather(i + 2, q)
        for p in range(4):
            if ds[p] is not None:
                ds[p].wait()
        return 0

    lax.fori_loop(0, N_BLK_E, blk, 0)
    plsc.subcore_barrier()
    pltpu.sync_copy(agg_sp.at[pl.ds(sid * ROWS_PER_TEC, ROWS_PER_TEC)],
                    out_hbm.at[cid, sid])


BLK_N = 2000


def _dense(x, comp_aug, bases_aug):
    r, nb = comp_aug.shape
    _, cin, cout = bases_aug.shape
    n = x.shape[0]

    def body(comp_ref, bases_ref, x_ref, out_ref, wcat_ref):
        @pl.when(pl.program_id(0) == 0)
        def _build():
            for rr in range(r):
                w = comp_ref[rr, 0] * bases_ref[0]
                for b in range(1, nb):
                    w = w + comp_ref[rr, b] * bases_ref[b]
                wcat_ref[:, rr * cout:(rr + 1) * cout] = w

        out_ref[...] = jnp.dot(x_ref[...], wcat_ref[...],
                               preferred_element_type=jnp.float32)

    return pl.pallas_call(
        body,
        grid=(n // BLK_N,),
        in_specs=[
            pl.BlockSpec((r, nb), lambda j: (0, 0),
                         memory_space=pltpu.SMEM),
            pl.BlockSpec((nb, cin, cout), lambda j: (0, 0, 0)),
            pl.BlockSpec((BLK_N, cin), lambda j: (j, 0)),
        ],
        out_specs=pl.BlockSpec((BLK_N, r * cout), lambda j: (j, 0)),
        out_shape=jax.ShapeDtypeStruct((n, r * cout), jnp.float32),
        scratch_shapes=[pltpu.VMEM((cin, r * cout), jnp.float32)],
    )(comp_aug, bases_aug, x)


def _combine(parts, xt_all, bias, relu):
    _, n, cout = parts.shape

    def body(parts_ref, xr_ref, bias_ref, out_ref):
        s = (parts_ref[0] + parts_ref[1] + xr_ref[:, :cout]
             + bias_ref[...])
        out_ref[...] = jnp.maximum(s, 0.0) if relu else s

    return pl.pallas_call(
        body,
        grid=(n // BLK_N,),
        in_specs=[
            pl.BlockSpec((NC, BLK_N, cout), lambda j: (0, j, 0)),
            pl.BlockSpec((BLK_N, 128), lambda j: (j, (N_REL * cout) // 128)),
            pl.BlockSpec((1, cout), lambda j: (0, 0)),
        ],
        out_specs=pl.BlockSpec((BLK_N, cout), lambda j: (j, 0)),
        out_shape=jax.ShapeDtypeStruct((n, cout), jnp.float32),
    )(parts, xt_all, bias.reshape(1, cout))


def _augment(comp, bases, root):
    nb = comp.shape[1]
    bases_aug = jnp.concatenate([bases, root[None]], axis=0)
    comp_aug = jnp.concatenate(
        [jnp.concatenate([comp, jnp.zeros((comp.shape[0], 1), comp.dtype)],
                         axis=1),
         jnp.zeros((1, nb + 1), comp.dtype).at[0, nb].set(1.0)],
        axis=0)
    return comp_aug, bases_aug


def kernel(x, bases1, comp1, root1, bias1, bases2, comp2, root2, bias2,
           edge_index, edge_type):
    src = edge_index[0]
    dst = edge_index[1]

    inv = _counts_inv(dst, edge_type)

    ca1, ba1 = _augment(comp1, bases1, root1)
    xt1 = _dense(x, ca1, ba1)
    d1 = bases1.shape[2]
    p1 = _edge_pass(xt1.reshape(N_NODES * (N_REL + 1), d1),
                    src, edge_type, inv, dst, d1)
    h = _combine(p1, xt1, bias1, relu=True)

    ca2, ba2 = _augment(comp2, bases2, root2)
    xt2 = _dense(h, ca2, ba2)
    d2 = bases2.shape[2]
    p2 = _edge_pass(xt2.reshape(N_NODES * (N_REL + 1), d2),
                    src, edge_type, inv, dst, d2)
    return _combine(p2, xt2, bias2, relu=False)
